# baseline (speedup 1.0000x reference)
import functools

import jax
import jax.numpy as jnp
from jax import lax
from jax.experimental import pallas as pl
from jax.experimental.pallas import tpu as pltpu
from jax.experimental.pallas import tpu_sc as plsc

N = 10000
NH = N // 2
E = 320000
D = 128

K = 128
EPT = 10240
E_PAD = 32 * EPT
LCH = EPT // K
NB = 4
ACC_ROWS = 5120
ZBPT = ACC_ROWS // 128 // 16 + 1
ORPT = 312


def _sc_filter_body(src_hbm, dst_hbm, slists_hbm, dlists_hbm, n4_hbm,
                    src_v, dst_v, slo_v, shi_v, dlo_v, dhi_v, n4_v):
    c = lax.axis_index("c")
    s = lax.axis_index("s")
    tid = s * 2 + c

    pltpu.sync_copy(src_hbm.at[tid], src_v)
    pltpu.sync_copy(dst_hbm.at[tid], dst_v)

    def _pf(i, carry):
        off = pl.ds(i * 16, 16)
        slo_v[off] = jnp.zeros((16,), jnp.int32)
        shi_v[off] = jnp.zeros((16,), jnp.int32)
        dlo_v[off] = jnp.full((16,), NH, jnp.int32)
        dhi_v[off] = jnp.full((16,), NH, jnp.int32)
        return carry
    lax.fori_loop(0, EPT // 16, _pf, 0)

    lane = lax.iota(jnp.int32, 16)

    def _flt(i, carry):
        cnt_lo, cnt_hi = carry
        sv = src_v[pl.ds(i * 16, 16)]
        dv = dst_v[pl.ds(i * 16, 16)]
        mlo = dv < NH
        mloi = jnp.where(mlo, 1, 0)
        incl = plsc.cumsum(mloi)
        excl = incl - mloi
        trash = EPT + lane
        idx_lo = jnp.where(mlo, cnt_lo + excl, trash)
        idx_hi = jnp.where(mlo, trash, cnt_hi + (lane - excl))
        plsc.store_scatter(slo_v, [idx_lo], sv)
        plsc.store_scatter(dlo_v, [idx_lo], dv)
        plsc.store_scatter(shi_v, [idx_hi], sv)
        plsc.store_scatter(dhi_v, [idx_hi], dv - NH)
        nlo = incl[15]
        return cnt_lo + nlo, cnt_hi + (16 - nlo)
    cnt_lo, cnt_hi = lax.fori_loop(0, EPT // 16, _flt, (0, 0))

    pltpu.sync_copy(slo_v.at[pl.ds(0, EPT)], slists_hbm.at[0, tid])
    pltpu.sync_copy(dlo_v.at[pl.ds(0, EPT)], dlists_hbm.at[0, tid])
    pltpu.sync_copy(shi_v.at[pl.ds(0, EPT)], slists_hbm.at[1, tid])
    pltpu.sync_copy(dhi_v.at[pl.ds(0, EPT)], dlists_hbm.at[1, tid])

    n4_v[pl.ds(0, 16)] = jnp.full((16,), (cnt_lo + NB * K - 1) // (NB * K),
                                  jnp.int32)
    pltpu.sync_copy(n4_v, n4_hbm.at[0, tid])
    n4_v[pl.ds(0, 16)] = jnp.full((16,), (cnt_hi + NB * K - 1) // (NB * K),
                                  jnp.int32)
    pltpu.sync_copy(n4_v, n4_hbm.at[1, tid])


@functools.cache
def _sc_filter():
    return pl.kernel(
        _sc_filter_body,
        out_type=[jax.ShapeDtypeStruct((2, 32, EPT), jnp.int32),
                  jax.ShapeDtypeStruct((2, 32, EPT), jnp.int32),
                  jax.ShapeDtypeStruct((2, 32, 16), jnp.int32)],
        mesh=plsc.VectorSubcoreMesh(core_axis_name="c", subcore_axis_name="s"),
        compiler_params=pltpu.CompilerParams(use_tc_tiling_on_sc=False,
                                             needs_layout_passes=False),
        scratch_types=[
            pltpu.VMEM((EPT,), jnp.int32),
            pltpu.VMEM((EPT,), jnp.int32),
            pltpu.VMEM((EPT + 16,), jnp.int32),
            pltpu.VMEM((EPT + 16,), jnp.int32),
            pltpu.VMEM((EPT + 16,), jnp.int32),
            pltpu.VMEM((EPT + 16,), jnp.int32),
            pltpu.VMEM((16,), jnp.int32),
        ],
    )


def _sc_segsum_body(u_hbm, slists_hbm, dlists_hbm, n4_hbm, out_hbm, *rest):
    src_v, dst_v = rest[0], rest[1]
    bufs = rest[2:2 + NB]
    n4_s = rest[2 + NB]
    acc_sh = rest[3 + NB]
    gsems = rest[4 + NB:4 + 2 * NB]
    ssems = rest[4 + 2 * NB:4 + 3 * NB]

    c = lax.axis_index("c")
    s = lax.axis_index("s")

    def _zb(i, carry):
        bufs[0][i // 8, pl.ds((i % 8) * 16, 16)] = jnp.zeros((16,), jnp.float32)
        return carry
    lax.fori_loop(0, 128 * 8, _zb, 0)

    for kblk in range(ZBPT):
        blk = s + 16 * kblk
        if kblk < ZBPT - 1:
            pltpu.sync_copy(bufs[0], acc_sh.at[pl.ds(blk * 128, 128)])
        else:
            @pl.when(blk < ACC_ROWS // 128)
            def _():
                pltpu.sync_copy(bufs[0], acc_sh.at[pl.ds(blk * 128, 128)])
    plsc.subcore_barrier()

    def _gwait(b):
        pltpu.make_async_copy(u_hbm.at[src_v.at[0]], bufs[b],
                              gsems[b]).wait()

    def _swait(b):
        pltpu.make_async_copy(bufs[b], acc_sh.at[dst_v.at[0]],
                              ssems[b]).wait()

    for t2 in (2 * s, 2 * s + 1):
        pltpu.sync_copy(slists_hbm.at[c, t2], src_v)
        pltpu.sync_copy(dlists_hbm.at[c, t2], dst_v)
        pltpu.sync_copy(n4_hbm.at[c, t2], n4_s)
        n4 = n4_s[pl.ds(0, 16)][0]
        nch = n4 * NB

        for b in range(NB - 1):
            @pl.when(b < nch)
            def _():
                pltpu.async_copy(u_hbm.at[src_v.at[b]], bufs[b], gsems[b])

        def _super(i, carry):
            for r in range(NB):
                j = i * NB + r
                q = (r + NB - 1) % NB

                @pl.when(j + NB - 1 < nch)
                def _():
                    @pl.when(j > 0)
                    def _():
                        _swait(q)
                    pltpu.async_copy(u_hbm.at[src_v.at[j + NB - 1]],
                                     bufs[q], gsems[q])

                _gwait(r)
                pltpu.async_copy(bufs[r], acc_sh.at[dst_v.at[j]],
                                 ssems[r], add=True)
            return carry
        lax.fori_loop(0, n4, _super, 0)

        for b in range(NB):
            @pl.when(b < nch)
            def _():
                _swait(b)

    plsc.subcore_barrier()
    base = c * NH
    pltpu.sync_copy(acc_sh.at[pl.ds(s * ORPT, ORPT)],
                    out_hbm.at[pl.ds(base + s * ORPT, ORPT)])

    @pl.when(s == 15)
    def _():
        tail = 16 * ORPT
        pltpu.sync_copy(acc_sh.at[pl.ds(tail, NH - tail)],
                        out_hbm.at[pl.ds(base + tail, NH - tail)])


@functools.cache
def _sc_segsum():
    return pl.kernel(
        _sc_segsum_body,
        out_type=jax.ShapeDtypeStruct((N, D), jnp.float32),
        mesh=plsc.VectorSubcoreMesh(core_axis_name="c", subcore_axis_name="s"),
        compiler_params=pltpu.CompilerParams(use_tc_tiling_on_sc=False),
        scratch_types=[
            pltpu.VMEM((LCH, K), jnp.int32),
            pltpu.VMEM((LCH, K), jnp.int32),
            *[pltpu.VMEM((K, D), jnp.float32) for _ in range(NB)],
            pltpu.VMEM((16,), jnp.int32),
            pltpu.VMEM_SHARED((ACC_ROWS, D), jnp.float32),
            *[pltpu.SemaphoreType.DMA for _ in range(2 * NB)],
        ],
    )


def _tc1_body(x_ref, h_ref, wa_ref, wb_ref, u_ref):
    u_ref[...] = (
        jnp.dot(x_ref[...], wa_ref[...], preferred_element_type=jnp.float32)
        + jnp.dot(h_ref[...], wb_ref[...], preferred_element_type=jnp.float32)
    )


def _tc2_body(x_ref, m_ref, wx_ref, b1_ref, w2_ref, b2_ref, o_ref):
    z = (jnp.dot(x_ref[...], wx_ref[...], preferred_element_type=jnp.float32)
         + m_ref[...] + b1_ref[...])
    hid = jnp.maximum(z, 0.0)
    o_ref[...] = (jnp.dot(hid, w2_ref[...], preferred_element_type=jnp.float32)
                  + b2_ref[...])


_ROWS_BLK = 1000


def kernel(x, h, edge_index, W1, b1, W2, b2):
    wx_t = W1[:, :D].T
    wa_t = W1[:, D:2 * D].T
    wb_t = W1[:, 2 * D:].T
    w2_t = W2.T

    grid = (N // _ROWS_BLK,)
    row_spec = pl.BlockSpec((_ROWS_BLK, D), lambda i: (i, 0))
    full_spec = pl.BlockSpec((D, D), lambda i: (0, 0))
    bias_spec = pl.BlockSpec((1, D), lambda i: (0, 0))

    u = pl.pallas_call(
        _tc1_body,
        grid=grid,
        in_specs=[row_spec, row_spec, full_spec, full_spec],
        out_specs=row_spec,
        out_shape=jax.ShapeDtypeStruct((N, D), jnp.float32),
    )(x, h, wa_t, wb_t)

    pad = E_PAD - E
    src2 = jnp.concatenate(
        [edge_index[0], jnp.zeros((pad,), jnp.int32)]).reshape(32, EPT)
    dst2 = jnp.concatenate(
        [edge_index[1], jnp.full((pad,), N, jnp.int32)]).reshape(32, EPT)

    slists, dlists, n4s = _sc_filter()(src2, dst2)
    m = _sc_segsum()(u, slists.reshape(2, 32, LCH, K),
                     dlists.reshape(2, 32, LCH, K), n4s)

    out = pl.pallas_call(
        _tc2_body,
        grid=grid,
        in_specs=[row_spec, row_spec, full_spec, bias_spec,
                  full_spec, bias_spec],
        out_specs=row_spec,
        out_shape=jax.ShapeDtypeStruct((N, D), jnp.float32),
    )(x, m, wx_t, b1.reshape(1, D), w2_t, b2.reshape(1, D))
    return out


# speedup vs baseline: 6.3263x; 6.3263x over previous
import functools

import jax
import jax.numpy as jnp
from jax import lax
from jax.experimental import pallas as pl
from jax.experimental.pallas import tpu as pltpu
from jax.experimental.pallas import tpu_sc as plsc

N = 10000
NH = N // 2
E = 320000
D = 128

K = 128
EPT = 10240
E_PAD = 32 * EPT
LCH = EPT // K
NB = 4
ACC_ROWS = 5120
ZBPT = ACC_ROWS // 128 // 16 + 1
ORPT = 312


def _sc_filter_body(src_hbm, dst_hbm, slists_hbm, dlists_hbm, n4_hbm,
                    src_v, dst_v, slo_v, shi_v, dlo_v, dhi_v, n4_v):
    c = lax.axis_index("c")
    s = lax.axis_index("s")
    tid = s * 2 + c

    pltpu.sync_copy(src_hbm.at[tid], src_v)
    pltpu.sync_copy(dst_hbm.at[tid], dst_v)

    lane0 = lax.iota(jnp.int32, 16)

    def _pf(i, carry):
        off = pl.ds(i * 16, 16)
        ent = i * 16 + lane0
        dummy_src = jnp.bitwise_and(ent, 8191)
        dummy_dst = NH + jnp.bitwise_and(ent, 63)
        slo_v[off] = dummy_src
        shi_v[off] = dummy_src
        dlo_v[off] = dummy_dst
        dhi_v[off] = dummy_dst
        return carry
    lax.fori_loop(0, EPT // 16, _pf, 0)

    lane = lax.iota(jnp.int32, 16)

    def _flt(i, carry):
        cnt_lo, cnt_hi = carry
        sv = src_v[pl.ds(i * 16, 16)]
        dv = dst_v[pl.ds(i * 16, 16)]
        mlo = dv < NH
        mloi = jnp.where(mlo, 1, 0)
        incl = plsc.cumsum(mloi)
        excl = incl - mloi
        trash = EPT + lane
        idx_lo = jnp.where(mlo, cnt_lo + excl, trash)
        idx_hi = jnp.where(mlo, trash, cnt_hi + (lane - excl))
        plsc.store_scatter(slo_v, [idx_lo], sv)
        plsc.store_scatter(dlo_v, [idx_lo], dv)
        plsc.store_scatter(shi_v, [idx_hi], sv)
        plsc.store_scatter(dhi_v, [idx_hi], dv - NH)
        nlo = incl[15]
        return cnt_lo + nlo, cnt_hi + (16 - nlo)
    cnt_lo, cnt_hi = lax.fori_loop(0, EPT // 16, _flt, (0, 0))

    pltpu.sync_copy(slo_v.at[pl.ds(0, EPT)], slists_hbm.at[0, tid])
    pltpu.sync_copy(dlo_v.at[pl.ds(0, EPT)], dlists_hbm.at[0, tid])
    pltpu.sync_copy(shi_v.at[pl.ds(0, EPT)], slists_hbm.at[1, tid])
    pltpu.sync_copy(dhi_v.at[pl.ds(0, EPT)], dlists_hbm.at[1, tid])

    n4_v[pl.ds(0, 16)] = jnp.full((16,), (cnt_lo + NB * K - 1) // (NB * K),
                                  jnp.int32)
    pltpu.sync_copy(n4_v, n4_hbm.at[0, tid])
    n4_v[pl.ds(0, 16)] = jnp.full((16,), (cnt_hi + NB * K - 1) // (NB * K),
                                  jnp.int32)
    pltpu.sync_copy(n4_v, n4_hbm.at[1, tid])


@functools.cache
def _sc_filter():
    return pl.kernel(
        _sc_filter_body,
        out_type=[jax.ShapeDtypeStruct((2, 32, EPT), jnp.int32),
                  jax.ShapeDtypeStruct((2, 32, EPT), jnp.int32),
                  jax.ShapeDtypeStruct((2, 32, 16), jnp.int32)],
        mesh=plsc.VectorSubcoreMesh(core_axis_name="c", subcore_axis_name="s"),
        compiler_params=pltpu.CompilerParams(use_tc_tiling_on_sc=False,
                                             needs_layout_passes=False),
        scratch_types=[
            pltpu.VMEM((EPT,), jnp.int32),
            pltpu.VMEM((EPT,), jnp.int32),
            pltpu.VMEM((EPT + 16,), jnp.int32),
            pltpu.VMEM((EPT + 16,), jnp.int32),
            pltpu.VMEM((EPT + 16,), jnp.int32),
            pltpu.VMEM((EPT + 16,), jnp.int32),
            pltpu.VMEM((16,), jnp.int32),
        ],
    )


def _sc_segsum_body(u_hbm, slists_hbm, dlists_hbm, n4_hbm, out_hbm, *rest):
    src_v, dst_v = rest[0], rest[1]
    bufs = rest[2:2 + NB]
    n4_s = rest[2 + NB]
    acc_sh = rest[3 + NB]
    gsems = rest[4 + NB:4 + 2 * NB]
    ssems = rest[4 + 2 * NB:4 + 3 * NB]

    c = lax.axis_index("c")
    s = lax.axis_index("s")

    def _zb(i, carry):
        bufs[0][i // 8, pl.ds((i % 8) * 16, 16)] = jnp.zeros((16,), jnp.float32)
        return carry
    lax.fori_loop(0, 128 * 8, _zb, 0)

    for kblk in range(ZBPT):
        blk = s + 16 * kblk
        if kblk < ZBPT - 1:
            pltpu.sync_copy(bufs[0], acc_sh.at[pl.ds(blk * 128, 128)])
        else:
            @pl.when(blk < ACC_ROWS // 128)
            def _():
                pltpu.sync_copy(bufs[0], acc_sh.at[pl.ds(blk * 128, 128)])
    plsc.subcore_barrier()

    def _gwait(b):
        pltpu.make_async_copy(u_hbm.at[src_v.at[0]], bufs[b],
                              gsems[b]).wait()

    def _swait(b):
        pltpu.make_async_copy(bufs[b], acc_sh.at[dst_v.at[0]],
                              ssems[b]).wait()

    for t2 in (2 * s, 2 * s + 1):
        pltpu.sync_copy(slists_hbm.at[c, t2], src_v)
        pltpu.sync_copy(dlists_hbm.at[c, t2], dst_v)
        pltpu.sync_copy(n4_hbm.at[c, t2], n4_s)
        n4 = n4_s[pl.ds(0, 16)][0]
        nch = n4 * NB

        for b in range(NB - 1):
            @pl.when(b < nch)
            def _():
                pltpu.async_copy(u_hbm.at[src_v.at[b]], bufs[b], gsems[b])

        def _super(i, carry):
            for r in range(NB):
                j = i * NB + r
                q = (r + NB - 1) % NB

                @pl.when(j + NB - 1 < nch)
                def _():
                    @pl.when(j > 0)
                    def _():
                        _swait(q)
                    pltpu.async_copy(u_hbm.at[src_v.at[j + NB - 1]],
                                     bufs[q], gsems[q])

                _gwait(r)
                pltpu.async_copy(bufs[r], acc_sh.at[dst_v.at[j]],
                                 ssems[r], add=True)
            return carry
        lax.fori_loop(0, n4, _super, 0)

        for b in range(NB):
            @pl.when(b < nch)
            def _():
                _swait(b)

    plsc.subcore_barrier()
    base = c * NH
    pltpu.sync_copy(acc_sh.at[pl.ds(s * ORPT, ORPT)],
                    out_hbm.at[pl.ds(base + s * ORPT, ORPT)])

    @pl.when(s == 15)
    def _():
        tail = 16 * ORPT
        pltpu.sync_copy(acc_sh.at[pl.ds(tail, NH - tail)],
                        out_hbm.at[pl.ds(base + tail, NH - tail)])


@functools.cache
def _sc_segsum():
    return pl.kernel(
        _sc_segsum_body,
        out_type=jax.ShapeDtypeStruct((N, D), jnp.float32),
        mesh=plsc.VectorSubcoreMesh(core_axis_name="c", subcore_axis_name="s"),
        compiler_params=pltpu.CompilerParams(use_tc_tiling_on_sc=False),
        scratch_types=[
            pltpu.VMEM((LCH, K), jnp.int32),
            pltpu.VMEM((LCH, K), jnp.int32),
            *[pltpu.VMEM((K, D), jnp.float32) for _ in range(NB)],
            pltpu.VMEM((16,), jnp.int32),
            pltpu.VMEM_SHARED((ACC_ROWS, D), jnp.float32),
            *[pltpu.SemaphoreType.DMA for _ in range(2 * NB)],
        ],
    )


def _tc1_body(x_ref, h_ref, wa_ref, wb_ref, u_ref):
    u_ref[...] = (
        jnp.dot(x_ref[...], wa_ref[...], preferred_element_type=jnp.float32)
        + jnp.dot(h_ref[...], wb_ref[...], preferred_element_type=jnp.float32)
    )


def _tc2_body(x_ref, m_ref, wx_ref, b1_ref, w2_ref, b2_ref, o_ref):
    z = (jnp.dot(x_ref[...], wx_ref[...], preferred_element_type=jnp.float32)
         + m_ref[...] + b1_ref[...])
    hid = jnp.maximum(z, 0.0)
    o_ref[...] = (jnp.dot(hid, w2_ref[...], preferred_element_type=jnp.float32)
                  + b2_ref[...])


_ROWS_BLK = 1000


def kernel(x, h, edge_index, W1, b1, W2, b2):
    wx_t = W1[:, :D].T
    wa_t = W1[:, D:2 * D].T
    wb_t = W1[:, 2 * D:].T
    w2_t = W2.T

    grid = (N // _ROWS_BLK,)
    row_spec = pl.BlockSpec((_ROWS_BLK, D), lambda i: (i, 0))
    full_spec = pl.BlockSpec((D, D), lambda i: (0, 0))
    bias_spec = pl.BlockSpec((1, D), lambda i: (0, 0))

    u = pl.pallas_call(
        _tc1_body,
        grid=grid,
        in_specs=[row_spec, row_spec, full_spec, full_spec],
        out_specs=row_spec,
        out_shape=jax.ShapeDtypeStruct((N, D), jnp.float32),
    )(x, h, wa_t, wb_t)

    pad = E_PAD - E
    pad_ar = jnp.arange(pad, dtype=jnp.int32)
    src2 = jnp.concatenate(
        [edge_index[0], pad_ar]).reshape(32, EPT)
    dst2 = jnp.concatenate(
        [edge_index[1], N + pad_ar % 64]).reshape(32, EPT)

    slists, dlists, n4s = _sc_filter()(src2, dst2)
    m = _sc_segsum()(u, slists.reshape(2, 32, LCH, K),
                     dlists.reshape(2, 32, LCH, K), n4s)

    out = pl.pallas_call(
        _tc2_body,
        grid=grid,
        in_specs=[row_spec, row_spec, full_spec, bias_spec,
                  full_spec, bias_spec],
        out_specs=row_spec,
        out_shape=jax.ShapeDtypeStruct((N, D), jnp.float32),
    )(x, m, wx_t, b1.reshape(1, D), w2_t, b2.reshape(1, D))
    return out


# speedup vs baseline: 7.7633x; 1.2271x over previous
import functools

import jax
import jax.numpy as jnp
from jax import lax
from jax.experimental import pallas as pl
from jax.experimental.pallas import tpu as pltpu
from jax.experimental.pallas import tpu_sc as plsc

N = 10000
NH = N // 2
E = 320000
D = 128

K = 128
EPT = 10240
EREAL = E // 32
LCH = EPT // K
NB = 4
ACC_ROWS = 5120
ZBPT = ACC_ROWS // 128 // 16 + 1
ORPT = 312


def _sc_filter_body(ei_hbm, slists_hbm, dlists_hbm, n4_hbm,
                    src_v, dst_v, slo_v, shi_v, dlo_v, dhi_v, n4_v):
    c = lax.axis_index("c")
    s = lax.axis_index("s")
    tid = s * 2 + c

    pltpu.sync_copy(ei_hbm.at[0, pl.ds(tid * EREAL, EREAL)], src_v)
    pltpu.sync_copy(ei_hbm.at[1, pl.ds(tid * EREAL, EREAL)], dst_v)

    lane0 = lax.iota(jnp.int32, 16)

    def _pf(i, carry):
        off = pl.ds(i * 16, 16)
        ent = i * 16 + lane0
        dummy_src = jnp.bitwise_and(ent, 8191)
        dummy_dst = NH + jnp.bitwise_and(ent, 63)
        slo_v[off] = dummy_src
        shi_v[off] = dummy_src
        dlo_v[off] = dummy_dst
        dhi_v[off] = dummy_dst
        return carry
    lax.fori_loop(0, EPT // 16, _pf, 0)

    lane = lax.iota(jnp.int32, 16)

    def _flt(i, carry):
        cnt_lo, cnt_hi = carry
        sv = src_v[pl.ds(i * 16, 16)]
        dv = dst_v[pl.ds(i * 16, 16)]
        mlo = dv < NH
        mloi = jnp.where(mlo, 1, 0)
        incl = plsc.cumsum(mloi)
        excl = incl - mloi
        trash = EPT + lane
        idx_lo = jnp.where(mlo, cnt_lo + excl, trash)
        idx_hi = jnp.where(mlo, trash, cnt_hi + (lane - excl))
        plsc.store_scatter(slo_v, [idx_lo], sv)
        plsc.store_scatter(dlo_v, [idx_lo], dv)
        plsc.store_scatter(shi_v, [idx_hi], sv)
        plsc.store_scatter(dhi_v, [idx_hi], dv - NH)
        nlo = incl[15]
        return cnt_lo + nlo, cnt_hi + (16 - nlo)
    cnt_lo, cnt_hi = lax.fori_loop(0, EREAL // 16, _flt, (0, 0))

    pltpu.sync_copy(slo_v.at[pl.ds(0, EPT)], slists_hbm.at[0, tid])
    pltpu.sync_copy(dlo_v.at[pl.ds(0, EPT)], dlists_hbm.at[0, tid])
    pltpu.sync_copy(shi_v.at[pl.ds(0, EPT)], slists_hbm.at[1, tid])
    pltpu.sync_copy(dhi_v.at[pl.ds(0, EPT)], dlists_hbm.at[1, tid])

    n4_v[pl.ds(0, 16)] = jnp.full((16,), (cnt_lo + NB * K - 1) // (NB * K),
                                  jnp.int32)
    pltpu.sync_copy(n4_v, n4_hbm.at[0, tid])
    n4_v[pl.ds(0, 16)] = jnp.full((16,), (cnt_hi + NB * K - 1) // (NB * K),
                                  jnp.int32)
    pltpu.sync_copy(n4_v, n4_hbm.at[1, tid])


@functools.cache
def _sc_filter():
    return pl.kernel(
        _sc_filter_body,
        out_type=[jax.ShapeDtypeStruct((2, 32, EPT), jnp.int32),
                  jax.ShapeDtypeStruct((2, 32, EPT), jnp.int32),
                  jax.ShapeDtypeStruct((2, 32, 16), jnp.int32)],
        mesh=plsc.VectorSubcoreMesh(core_axis_name="c", subcore_axis_name="s"),
        compiler_params=pltpu.CompilerParams(use_tc_tiling_on_sc=False,
                                             needs_layout_passes=False),
        scratch_types=[
            pltpu.VMEM((EREAL,), jnp.int32),
            pltpu.VMEM((EREAL,), jnp.int32),
            pltpu.VMEM((EPT + 16,), jnp.int32),
            pltpu.VMEM((EPT + 16,), jnp.int32),
            pltpu.VMEM((EPT + 16,), jnp.int32),
            pltpu.VMEM((EPT + 16,), jnp.int32),
            pltpu.VMEM((16,), jnp.int32),
        ],
    )


def _sc_segsum_body(u_hbm, slists_hbm, dlists_hbm, n4_hbm, out_hbm, *rest):
    src_v, dst_v = rest[0], rest[1]
    bufs = rest[2:2 + NB]
    n4_s = rest[2 + NB]
    acc_sh = rest[3 + NB]
    gsems = rest[4 + NB:4 + 2 * NB]
    ssems = rest[4 + 2 * NB:4 + 3 * NB]

    c = lax.axis_index("c")
    s = lax.axis_index("s")

    def _zb(i, carry):
        bufs[0][i // 8, pl.ds((i % 8) * 16, 16)] = jnp.zeros((16,), jnp.float32)
        return carry
    lax.fori_loop(0, 128 * 8, _zb, 0)

    for kblk in range(ZBPT):
        blk = s + 16 * kblk
        if kblk < ZBPT - 1:
            pltpu.sync_copy(bufs[0], acc_sh.at[pl.ds(blk * 128, 128)])
        else:
            @pl.when(blk < ACC_ROWS // 128)
            def _():
                pltpu.sync_copy(bufs[0], acc_sh.at[pl.ds(blk * 128, 128)])
    plsc.subcore_barrier()

    def _gwait(b):
        pltpu.make_async_copy(u_hbm.at[src_v.at[0]], bufs[b],
                              gsems[b]).wait()

    def _swait(b):
        pltpu.make_async_copy(bufs[b], acc_sh.at[dst_v.at[0]],
                              ssems[b]).wait()

    for t2 in (2 * s, 2 * s + 1):
        pltpu.sync_copy(slists_hbm.at[c, t2], src_v)
        pltpu.sync_copy(dlists_hbm.at[c, t2], dst_v)
        pltpu.sync_copy(n4_hbm.at[c, t2], n4_s)
        n4 = n4_s[pl.ds(0, 16)][0]
        nch = n4 * NB

        for b in range(NB - 1):
            @pl.when(b < nch)
            def _():
                pltpu.async_copy(u_hbm.at[src_v.at[b]], bufs[b], gsems[b])

        def _super(i, carry):
            for r in range(NB):
                j = i * NB + r
                q = (r + NB - 1) % NB

                @pl.when(j + NB - 1 < nch)
                def _():
                    @pl.when(j > 0)
                    def _():
                        _swait(q)
                    pltpu.async_copy(u_hbm.at[src_v.at[j + NB - 1]],
                                     bufs[q], gsems[q])

                _gwait(r)
                pltpu.async_copy(bufs[r], acc_sh.at[dst_v.at[j]],
                                 ssems[r], add=True)
            return carry
        lax.fori_loop(0, n4, _super, 0)

        for b in range(NB):
            @pl.when(b < nch)
            def _():
                _swait(b)

    plsc.subcore_barrier()
    base = c * NH
    pltpu.sync_copy(acc_sh.at[pl.ds(s * ORPT, ORPT)],
                    out_hbm.at[pl.ds(base + s * ORPT, ORPT)])

    @pl.when(s == 15)
    def _():
        tail = 16 * ORPT
        pltpu.sync_copy(acc_sh.at[pl.ds(tail, NH - tail)],
                        out_hbm.at[pl.ds(base + tail, NH - tail)])


@functools.cache
def _sc_segsum():
    return pl.kernel(
        _sc_segsum_body,
        out_type=jax.ShapeDtypeStruct((N, D), jnp.float32),
        mesh=plsc.VectorSubcoreMesh(core_axis_name="c", subcore_axis_name="s"),
        compiler_params=pltpu.CompilerParams(use_tc_tiling_on_sc=False),
        scratch_types=[
            pltpu.VMEM((LCH, K), jnp.int32),
            pltpu.VMEM((LCH, K), jnp.int32),
            *[pltpu.VMEM((K, D), jnp.float32) for _ in range(NB)],
            pltpu.VMEM((16,), jnp.int32),
            pltpu.VMEM_SHARED((ACC_ROWS, D), jnp.float32),
            *[pltpu.SemaphoreType.DMA for _ in range(2 * NB)],
        ],
    )


def _tc1_body(x_ref, h_ref, wa_ref, wb_ref, u_ref):
    u_ref[...] = (
        jnp.dot(x_ref[...], wa_ref[...], preferred_element_type=jnp.float32)
        + jnp.dot(h_ref[...], wb_ref[...], preferred_element_type=jnp.float32)
    )


def _tc2_body(x_ref, m_ref, wx_ref, b1_ref, w2_ref, b2_ref, o_ref):
    z = (jnp.dot(x_ref[...], wx_ref[...], preferred_element_type=jnp.float32)
         + m_ref[...] + b1_ref[...])
    hid = jnp.maximum(z, 0.0)
    o_ref[...] = (jnp.dot(hid, w2_ref[...], preferred_element_type=jnp.float32)
                  + b2_ref[...])


_ROWS_BLK = 2000


def kernel(x, h, edge_index, W1, b1, W2, b2):
    wx_t = W1[:, :D].T
    wa_t = W1[:, D:2 * D].T
    wb_t = W1[:, 2 * D:].T
    w2_t = W2.T

    grid = (N // _ROWS_BLK,)
    row_spec = pl.BlockSpec((_ROWS_BLK, D), lambda i: (i, 0))
    full_spec = pl.BlockSpec((D, D), lambda i: (0, 0))
    bias_spec = pl.BlockSpec((1, D), lambda i: (0, 0))

    u = pl.pallas_call(
        _tc1_body,
        grid=grid,
        in_specs=[row_spec, row_spec, full_spec, full_spec],
        out_specs=row_spec,
        out_shape=jax.ShapeDtypeStruct((N, D), jnp.float32),
    )(x, h, wa_t, wb_t)

    slists, dlists, n4s = _sc_filter()(edge_index)
    m = _sc_segsum()(u, slists.reshape(2, 32, LCH, K),
                     dlists.reshape(2, 32, LCH, K), n4s)

    out = pl.pallas_call(
        _tc2_body,
        grid=grid,
        in_specs=[row_spec, row_spec, full_spec, bias_spec,
                  full_spec, bias_spec],
        out_specs=row_spec,
        out_shape=jax.ShapeDtypeStruct((N, D), jnp.float32),
    )(x, m, wx_t, b1.reshape(1, D), w2_t, b2.reshape(1, D))
    return out


# speedup vs baseline: 7.7725x; 1.0012x over previous
import functools

import jax
import jax.numpy as jnp
from jax import lax
from jax.experimental import pallas as pl
from jax.experimental.pallas import tpu as pltpu
from jax.experimental.pallas import tpu_sc as plsc

N = 10000
NH = N // 2
E = 320000
D = 128

K = 128
EPT = 10240
EREAL = E // 32
LCH = EPT // K
NB = 4
ACC_ROWS = 5120
ZBPT = ACC_ROWS // 128 // 16 + 1
ORPT = 312


def _sc_filter_body(ei_hbm, slists_hbm, dlists_hbm, n4_hbm,
                    src_v, dst_v, slo_v, shi_v, dlo_v, dhi_v, n4_v):
    c = lax.axis_index("c")
    s = lax.axis_index("s")
    tid = s * 2 + c

    pltpu.sync_copy(ei_hbm.at[0, pl.ds(tid * EREAL, EREAL)], src_v)
    pltpu.sync_copy(ei_hbm.at[1, pl.ds(tid * EREAL, EREAL)], dst_v)

    lane0 = lax.iota(jnp.int32, 16)

    def _pf(i, carry):
        off = pl.ds(i * 16, 16)
        ent = i * 16 + lane0
        dummy_src = jnp.bitwise_and(ent, 8191)
        dummy_dst = NH + jnp.bitwise_and(ent, 63)
        slo_v[off] = dummy_src
        shi_v[off] = dummy_src
        dlo_v[off] = dummy_dst
        dhi_v[off] = dummy_dst
        return carry
    lax.fori_loop(0, EPT // 16, _pf, 0)

    lane = lax.iota(jnp.int32, 16)

    def _flt(i, carry):
        cnt_lo, cnt_hi = carry
        sv = src_v[pl.ds(i * 16, 16)]
        dv = dst_v[pl.ds(i * 16, 16)]
        mlo = dv < NH
        mloi = jnp.where(mlo, 1, 0)
        incl = plsc.cumsum(mloi)
        excl = incl - mloi
        trash = EPT + NB * K + lane
        idx_lo = jnp.where(mlo, cnt_lo + excl, trash)
        idx_hi = jnp.where(mlo, trash, cnt_hi + (lane - excl))
        plsc.store_scatter(slo_v, [idx_lo], sv)
        plsc.store_scatter(dlo_v, [idx_lo], dv)
        plsc.store_scatter(shi_v, [idx_hi], sv)
        plsc.store_scatter(dhi_v, [idx_hi], dv - NH)
        nlo = incl[15]
        return cnt_lo + nlo, cnt_hi + (16 - nlo)
    cnt_lo, cnt_hi = lax.fori_loop(0, EREAL // 16, _flt, (0, 0))

    def _pf(i, carry):
        ent = i * 16 + lane
        dummy_src = jnp.bitwise_and(ent, 8191)
        dummy_dst = NH + jnp.bitwise_and(ent, 63)
        plsc.store_scatter(slo_v, [cnt_lo + ent], dummy_src)
        plsc.store_scatter(dlo_v, [cnt_lo + ent], dummy_dst)
        plsc.store_scatter(shi_v, [cnt_hi + ent], dummy_src)
        plsc.store_scatter(dhi_v, [cnt_hi + ent], dummy_dst)
        return carry
    lax.fori_loop(0, (NB * K) // 16, _pf, 0)

    pltpu.sync_copy(slo_v.at[pl.ds(0, EPT)], slists_hbm.at[0, tid])
    pltpu.sync_copy(dlo_v.at[pl.ds(0, EPT)], dlists_hbm.at[0, tid])
    pltpu.sync_copy(shi_v.at[pl.ds(0, EPT)], slists_hbm.at[1, tid])
    pltpu.sync_copy(dhi_v.at[pl.ds(0, EPT)], dlists_hbm.at[1, tid])

    n4_v[pl.ds(0, 16)] = jnp.full((16,), (cnt_lo + NB * K - 1) // (NB * K),
                                  jnp.int32)
    pltpu.sync_copy(n4_v, n4_hbm.at[0, tid])
    n4_v[pl.ds(0, 16)] = jnp.full((16,), (cnt_hi + NB * K - 1) // (NB * K),
                                  jnp.int32)
    pltpu.sync_copy(n4_v, n4_hbm.at[1, tid])


@functools.cache
def _sc_filter():
    return pl.kernel(
        _sc_filter_body,
        out_type=[jax.ShapeDtypeStruct((2, 32, EPT), jnp.int32),
                  jax.ShapeDtypeStruct((2, 32, EPT), jnp.int32),
                  jax.ShapeDtypeStruct((2, 32, 16), jnp.int32)],
        mesh=plsc.VectorSubcoreMesh(core_axis_name="c", subcore_axis_name="s"),
        compiler_params=pltpu.CompilerParams(use_tc_tiling_on_sc=False,
                                             needs_layout_passes=False),
        scratch_types=[
            pltpu.VMEM((EREAL,), jnp.int32),
            pltpu.VMEM((EREAL,), jnp.int32),
            pltpu.VMEM((EPT + NB * K + 16,), jnp.int32),
            pltpu.VMEM((EPT + NB * K + 16,), jnp.int32),
            pltpu.VMEM((EPT + NB * K + 16,), jnp.int32),
            pltpu.VMEM((EPT + NB * K + 16,), jnp.int32),
            pltpu.VMEM((16,), jnp.int32),
        ],
    )


def _sc_segsum_body(u_hbm, slists_hbm, dlists_hbm, n4_hbm, out_hbm, *rest):
    src_v, dst_v = rest[0], rest[1]
    bufs = rest[2:2 + NB]
    n4_s = rest[2 + NB]
    acc_sh = rest[3 + NB]
    gsems = rest[4 + NB:4 + 2 * NB]
    ssems = rest[4 + 2 * NB:4 + 3 * NB]

    c = lax.axis_index("c")
    s = lax.axis_index("s")

    def _zb(i, carry):
        bufs[0][i // 8, pl.ds((i % 8) * 16, 16)] = jnp.zeros((16,), jnp.float32)
        return carry
    lax.fori_loop(0, 128 * 8, _zb, 0)

    for kblk in range(ZBPT):
        blk = s + 16 * kblk
        if kblk < ZBPT - 1:
            pltpu.sync_copy(bufs[0], acc_sh.at[pl.ds(blk * 128, 128)])
        else:
            @pl.when(blk < ACC_ROWS // 128)
            def _():
                pltpu.sync_copy(bufs[0], acc_sh.at[pl.ds(blk * 128, 128)])
    plsc.subcore_barrier()

    def _gwait(b):
        pltpu.make_async_copy(u_hbm.at[src_v.at[0]], bufs[b],
                              gsems[b]).wait()

    def _swait(b):
        pltpu.make_async_copy(bufs[b], acc_sh.at[dst_v.at[0]],
                              ssems[b]).wait()

    for t2 in (2 * s, 2 * s + 1):
        pltpu.sync_copy(slists_hbm.at[c, t2], src_v)
        pltpu.sync_copy(dlists_hbm.at[c, t2], dst_v)
        pltpu.sync_copy(n4_hbm.at[c, t2], n4_s)
        n4 = n4_s[pl.ds(0, 16)][0]
        nch = n4 * NB

        for b in range(NB - 1):
            @pl.when(b < nch)
            def _():
                pltpu.async_copy(u_hbm.at[src_v.at[b]], bufs[b], gsems[b])

        def _super(i, carry):
            for r in range(NB):
                j = i * NB + r
                q = (r + NB - 1) % NB

                @pl.when(j + NB - 1 < nch)
                def _():
                    @pl.when(j > 0)
                    def _():
                        _swait(q)
                    pltpu.async_copy(u_hbm.at[src_v.at[j + NB - 1]],
                                     bufs[q], gsems[q])

                _gwait(r)
                pltpu.async_copy(bufs[r], acc_sh.at[dst_v.at[j]],
                                 ssems[r], add=True)
            return carry
        lax.fori_loop(0, n4, _super, 0)

        for b in range(NB):
            @pl.when(b < nch)
            def _():
                _swait(b)

    plsc.subcore_barrier()
    base = c * NH
    pltpu.sync_copy(acc_sh.at[pl.ds(s * ORPT, ORPT)],
                    out_hbm.at[pl.ds(base + s * ORPT, ORPT)])

    @pl.when(s == 15)
    def _():
        tail = 16 * ORPT
        pltpu.sync_copy(acc_sh.at[pl.ds(tail, NH - tail)],
                        out_hbm.at[pl.ds(base + tail, NH - tail)])


@functools.cache
def _sc_segsum():
    return pl.kernel(
        _sc_segsum_body,
        out_type=jax.ShapeDtypeStruct((N, D), jnp.float32),
        mesh=plsc.VectorSubcoreMesh(core_axis_name="c", subcore_axis_name="s"),
        compiler_params=pltpu.CompilerParams(use_tc_tiling_on_sc=False),
        scratch_types=[
            pltpu.VMEM((LCH, K), jnp.int32),
            pltpu.VMEM((LCH, K), jnp.int32),
            *[pltpu.VMEM((K, D), jnp.float32) for _ in range(NB)],
            pltpu.VMEM((16,), jnp.int32),
            pltpu.VMEM_SHARED((ACC_ROWS, D), jnp.float32),
            *[pltpu.SemaphoreType.DMA for _ in range(2 * NB)],
        ],
    )


def _tc1_body(x_ref, h_ref, wa_ref, wb_ref, u_ref):
    u_ref[...] = (
        jnp.dot(x_ref[...], wa_ref[...], preferred_element_type=jnp.float32)
        + jnp.dot(h_ref[...], wb_ref[...], preferred_element_type=jnp.float32)
    )


def _tc2_body(x_ref, m_ref, wx_ref, b1_ref, w2_ref, b2_ref, o_ref):
    z = (jnp.dot(x_ref[...], wx_ref[...], preferred_element_type=jnp.float32)
         + m_ref[...] + b1_ref[...])
    hid = jnp.maximum(z, 0.0)
    o_ref[...] = (jnp.dot(hid, w2_ref[...], preferred_element_type=jnp.float32)
                  + b2_ref[...])


_ROWS_BLK = 2000


def kernel(x, h, edge_index, W1, b1, W2, b2):
    wx_t = W1[:, :D].T
    wa_t = W1[:, D:2 * D].T
    wb_t = W1[:, 2 * D:].T
    w2_t = W2.T

    grid = (N // _ROWS_BLK,)
    row_spec = pl.BlockSpec((_ROWS_BLK, D), lambda i: (i, 0))
    full_spec = pl.BlockSpec((D, D), lambda i: (0, 0))
    bias_spec = pl.BlockSpec((1, D), lambda i: (0, 0))

    u = pl.pallas_call(
        _tc1_body,
        grid=grid,
        in_specs=[row_spec, row_spec, full_spec, full_spec],
        out_specs=row_spec,
        out_shape=jax.ShapeDtypeStruct((N, D), jnp.float32),
    )(x, h, wa_t, wb_t)

    slists, dlists, n4s = _sc_filter()(edge_index)
    m = _sc_segsum()(u, slists.reshape(2, 32, LCH, K),
                     dlists.reshape(2, 32, LCH, K), n4s)

    out = pl.pallas_call(
        _tc2_body,
        grid=grid,
        in_specs=[row_spec, row_spec, full_spec, bias_spec,
                  full_spec, bias_spec],
        out_specs=row_spec,
        out_shape=jax.ShapeDtypeStruct((N, D), jnp.float32),
    )(x, m, wx_t, b1.reshape(1, D), w2_t, b2.reshape(1, D))
    return out


# speedup vs baseline: 7.7924x; 1.0026x over previous
import functools

import jax
import jax.numpy as jnp
from jax import lax
from jax.experimental import pallas as pl
from jax.experimental.pallas import tpu as pltpu
from jax.experimental.pallas import tpu_sc as plsc

N = 10000
NH = N // 2
E = 320000
D = 128

K = 128
EPT = 10240
EREAL = E // 32
LCH = EPT // K
NB = 4
ACC_ROWS = 5120
ZBPT = ACC_ROWS // 128 // 16 + 1
ORPT = 312


def _sc_filter_body(ei_hbm, slists_hbm, dlists_hbm, n4_hbm,
                    src_v, dst_v, slo_v, shi_v, dlo_v, dhi_v, n4_v, osem):
    c = lax.axis_index("c")
    s = lax.axis_index("s")
    tid = s * 2 + c

    pltpu.sync_copy(ei_hbm.at[0, pl.ds(tid * EREAL, EREAL)], src_v)
    pltpu.sync_copy(ei_hbm.at[1, pl.ds(tid * EREAL, EREAL)], dst_v)

    lane0 = lax.iota(jnp.int32, 16)

    def _pf(i, carry):
        off = pl.ds(i * 16, 16)
        ent = i * 16 + lane0
        dummy_src = jnp.bitwise_and(ent, 8191)
        dummy_dst = NH + jnp.bitwise_and(ent, 63)
        slo_v[off] = dummy_src
        shi_v[off] = dummy_src
        dlo_v[off] = dummy_dst
        dhi_v[off] = dummy_dst
        return carry
    lax.fori_loop(0, EPT // 16, _pf, 0)

    lane = lax.iota(jnp.int32, 16)

    def _flt(i, carry):
        cnt_lo, cnt_hi = carry
        sv = src_v[pl.ds(i * 16, 16)]
        dv = dst_v[pl.ds(i * 16, 16)]
        mlo = dv < NH
        mloi = jnp.where(mlo, 1, 0)
        incl = plsc.cumsum(mloi)
        excl = incl - mloi
        trash = EPT + NB * K + lane
        idx_lo = jnp.where(mlo, cnt_lo + excl, trash)
        idx_hi = jnp.where(mlo, trash, cnt_hi + (lane - excl))
        plsc.store_scatter(slo_v, [idx_lo], sv)
        plsc.store_scatter(dlo_v, [idx_lo], dv)
        plsc.store_scatter(shi_v, [idx_hi], sv)
        plsc.store_scatter(dhi_v, [idx_hi], dv - NH)
        nlo = incl[15]
        return cnt_lo + nlo, cnt_hi + (16 - nlo)
    cnt_lo, cnt_hi = lax.fori_loop(0, EREAL // 16, _flt, (0, 0))

    def _pf(i, carry):
        ent = i * 16 + lane
        dummy_src = jnp.bitwise_and(ent, 8191)
        dummy_dst = NH + jnp.bitwise_and(ent, 63)
        plsc.store_scatter(slo_v, [cnt_lo + ent], dummy_src)
        plsc.store_scatter(dlo_v, [cnt_lo + ent], dummy_dst)
        plsc.store_scatter(shi_v, [cnt_hi + ent], dummy_src)
        plsc.store_scatter(dhi_v, [cnt_hi + ent], dummy_dst)
        return carry
    lax.fori_loop(0, (NB * K) // 16, _pf, 0)

    cp1 = pltpu.async_copy(slo_v.at[pl.ds(0, EPT)], slists_hbm.at[0, tid],
                           osem)
    cp2 = pltpu.async_copy(dlo_v.at[pl.ds(0, EPT)], dlists_hbm.at[0, tid],
                           osem)
    cp3 = pltpu.async_copy(shi_v.at[pl.ds(0, EPT)], slists_hbm.at[1, tid],
                           osem)
    cp4 = pltpu.async_copy(dhi_v.at[pl.ds(0, EPT)], dlists_hbm.at[1, tid],
                           osem)

    n4_v[pl.ds(0, 16)] = jnp.full((16,), (cnt_lo + NB * K - 1) // (NB * K),
                                  jnp.int32)
    pltpu.sync_copy(n4_v, n4_hbm.at[0, tid])
    n4_v[pl.ds(0, 16)] = jnp.full((16,), (cnt_hi + NB * K - 1) // (NB * K),
                                  jnp.int32)
    pltpu.sync_copy(n4_v, n4_hbm.at[1, tid])
    cp1.wait()
    cp2.wait()
    cp3.wait()
    cp4.wait()


@functools.cache
def _sc_filter():
    return pl.kernel(
        _sc_filter_body,
        out_type=[jax.ShapeDtypeStruct((2, 32, EPT), jnp.int32),
                  jax.ShapeDtypeStruct((2, 32, EPT), jnp.int32),
                  jax.ShapeDtypeStruct((2, 32, 16), jnp.int32)],
        mesh=plsc.VectorSubcoreMesh(core_axis_name="c", subcore_axis_name="s"),
        compiler_params=pltpu.CompilerParams(use_tc_tiling_on_sc=False,
                                             needs_layout_passes=False),
        scratch_types=[
            pltpu.VMEM((EREAL,), jnp.int32),
            pltpu.VMEM((EREAL,), jnp.int32),
            pltpu.VMEM((EPT + NB * K + 16,), jnp.int32),
            pltpu.VMEM((EPT + NB * K + 16,), jnp.int32),
            pltpu.VMEM((EPT + NB * K + 16,), jnp.int32),
            pltpu.VMEM((EPT + NB * K + 16,), jnp.int32),
            pltpu.VMEM((16,), jnp.int32),
            pltpu.SemaphoreType.DMA,
        ],
    )


def _sc_segsum_body(u_hbm, slists_hbm, dlists_hbm, n4_hbm, out_hbm, *rest):
    src_v, dst_v = rest[0], rest[1]
    bufs = rest[2:2 + NB]
    n4_s = rest[2 + NB]
    acc_sh = rest[3 + NB]
    gsems = rest[4 + NB:4 + 2 * NB]
    ssems = rest[4 + 2 * NB:4 + 3 * NB]

    c = lax.axis_index("c")
    s = lax.axis_index("s")

    def _zb(i, carry):
        bufs[0][i // 8, pl.ds((i % 8) * 16, 16)] = jnp.zeros((16,), jnp.float32)
        return carry
    lax.fori_loop(0, 128 * 8, _zb, 0)

    for kblk in range(ZBPT):
        blk = s + 16 * kblk
        if kblk < ZBPT - 1:
            pltpu.sync_copy(bufs[0], acc_sh.at[pl.ds(blk * 128, 128)])
        else:
            @pl.when(blk < ACC_ROWS // 128)
            def _():
                pltpu.sync_copy(bufs[0], acc_sh.at[pl.ds(blk * 128, 128)])
    plsc.subcore_barrier()

    def _gwait(b):
        pltpu.make_async_copy(u_hbm.at[src_v.at[0]], bufs[b],
                              gsems[b]).wait()

    def _swait(b):
        pltpu.make_async_copy(bufs[b], acc_sh.at[dst_v.at[0]],
                              ssems[b]).wait()

    for t2 in (2 * s, 2 * s + 1):
        pltpu.sync_copy(slists_hbm.at[c, t2], src_v)
        pltpu.sync_copy(dlists_hbm.at[c, t2], dst_v)
        pltpu.sync_copy(n4_hbm.at[c, t2], n4_s)
        n4 = n4_s[pl.ds(0, 16)][0]
        nch = n4 * NB

        for b in range(NB - 1):
            @pl.when(b < nch)
            def _():
                pltpu.async_copy(u_hbm.at[src_v.at[b]], bufs[b], gsems[b])

        def _super(i, carry):
            for r in range(NB):
                j = i * NB + r
                q = (r + NB - 1) % NB

                @pl.when(j + NB - 1 < nch)
                def _():
                    @pl.when(j > 0)
                    def _():
                        _swait(q)
                    pltpu.async_copy(u_hbm.at[src_v.at[j + NB - 1]],
                                     bufs[q], gsems[q])

                _gwait(r)
                pltpu.async_copy(bufs[r], acc_sh.at[dst_v.at[j]],
                                 ssems[r], add=True)
            return carry
        lax.fori_loop(0, n4, _super, 0)

        for b in range(NB):
            @pl.when(b < nch)
            def _():
                _swait(b)

    plsc.subcore_barrier()
    base = c * NH
    pltpu.sync_copy(acc_sh.at[pl.ds(s * ORPT, ORPT)],
                    out_hbm.at[pl.ds(base + s * ORPT, ORPT)])

    @pl.when(s == 15)
    def _():
        tail = 16 * ORPT
        pltpu.sync_copy(acc_sh.at[pl.ds(tail, NH - tail)],
                        out_hbm.at[pl.ds(base + tail, NH - tail)])


@functools.cache
def _sc_segsum():
    return pl.kernel(
        _sc_segsum_body,
        out_type=jax.ShapeDtypeStruct((N, D), jnp.float32),
        mesh=plsc.VectorSubcoreMesh(core_axis_name="c", subcore_axis_name="s"),
        compiler_params=pltpu.CompilerParams(use_tc_tiling_on_sc=False),
        scratch_types=[
            pltpu.VMEM((LCH, K), jnp.int32),
            pltpu.VMEM((LCH, K), jnp.int32),
            *[pltpu.VMEM((K, D), jnp.float32) for _ in range(NB)],
            pltpu.VMEM((16,), jnp.int32),
            pltpu.VMEM_SHARED((ACC_ROWS, D), jnp.float32),
            *[pltpu.SemaphoreType.DMA for _ in range(2 * NB)],
        ],
    )


def _tc1_body(x_ref, h_ref, wa_ref, wb_ref, u_ref):
    u_ref[...] = (
        jnp.dot(x_ref[...], wa_ref[...], preferred_element_type=jnp.float32)
        + jnp.dot(h_ref[...], wb_ref[...], preferred_element_type=jnp.float32)
    )


def _tc2_body(x_ref, m_ref, wx_ref, b1_ref, w2_ref, b2_ref, o_ref):
    z = (jnp.dot(x_ref[...], wx_ref[...], preferred_element_type=jnp.float32)
         + m_ref[...] + b1_ref[...])
    hid = jnp.maximum(z, 0.0)
    o_ref[...] = (jnp.dot(hid, w2_ref[...], preferred_element_type=jnp.float32)
                  + b2_ref[...])


_ROWS_BLK = 2000


def kernel(x, h, edge_index, W1, b1, W2, b2):
    wx_t = W1[:, :D].T
    wa_t = W1[:, D:2 * D].T
    wb_t = W1[:, 2 * D:].T
    w2_t = W2.T

    grid = (N // _ROWS_BLK,)
    row_spec = pl.BlockSpec((_ROWS_BLK, D), lambda i: (i, 0))
    full_spec = pl.BlockSpec((D, D), lambda i: (0, 0))
    bias_spec = pl.BlockSpec((1, D), lambda i: (0, 0))

    u = pl.pallas_call(
        _tc1_body,
        grid=grid,
        in_specs=[row_spec, row_spec, full_spec, full_spec],
        out_specs=row_spec,
        out_shape=jax.ShapeDtypeStruct((N, D), jnp.float32),
    )(x, h, wa_t, wb_t)

    slists, dlists, n4s = _sc_filter()(edge_index)
    m = _sc_segsum()(u, slists.reshape(2, 32, LCH, K),
                     dlists.reshape(2, 32, LCH, K), n4s)

    out = pl.pallas_call(
        _tc2_body,
        grid=grid,
        in_specs=[row_spec, row_spec, full_spec, bias_spec,
                  full_spec, bias_spec],
        out_specs=row_spec,
        out_shape=jax.ShapeDtypeStruct((N, D), jnp.float32),
    )(x, m, wx_t, b1.reshape(1, D), w2_t, b2.reshape(1, D))
    return out
